# baseline (speedup 1.0000x reference)
import functools

import jax
import jax.numpy as jnp
from jax import lax
from jax.experimental import pallas as pl
from jax.experimental.pallas import tpu as pltpu
from jax.experimental.pallas import tpu_sc as plsc

SEQ = 32
VOCAB = 1_000_000
BATCH = 32
CHUNK = 65536
NCHUNK = -(-VOCAB // CHUNK)


def _lse_body(x_ref, out_ref, m_ref, s_ref):
    i = pl.program_id(0)
    x = x_ref[...]
    col = lax.broadcasted_iota(jnp.int32, (SEQ, CHUNK), 1) + i * CHUNK
    x = jnp.where(col < VOCAB, x, -jnp.inf)
    mc = jnp.max(x, axis=1, keepdims=True)

    @pl.when(i == 0)
    def _():
        m_ref[...] = jnp.broadcast_to(mc, (SEQ, 128))
        sc = jnp.sum(jnp.exp(x - mc), axis=1, keepdims=True)
        s_ref[...] = jnp.broadcast_to(sc, (SEQ, 128))

    @pl.when(i > 0)
    def _():
        m_old = m_ref[:, :1]
        m_new = jnp.maximum(m_old, mc)
        s_new = s_ref[:, :1] * jnp.exp(m_old - m_new) + jnp.sum(
            jnp.exp(x - m_new), axis=1, keepdims=True
        )
        m_ref[...] = jnp.broadcast_to(m_new, (SEQ, 128))
        s_ref[...] = jnp.broadcast_to(s_new, (SEQ, 128))

    @pl.when(i == NCHUNK - 1)
    def _():
        out_ref[...] = m_ref[...] + jnp.log(s_ref[...])


def _lse(logits):
    out = pl.pallas_call(
        _lse_body,
        grid=(NCHUNK,),
        in_specs=[pl.BlockSpec((SEQ, CHUNK), lambda i: (0, i))],
        out_specs=pl.BlockSpec((SEQ, 128), lambda i: (0, 0)),
        out_shape=jax.ShapeDtypeStruct((SEQ, 128), jnp.float32),
        scratch_shapes=[
            pltpu.VMEM((SEQ, 128), jnp.float32),
            pltpu.VMEM((SEQ, 128), jnp.float32),
        ],
    )(logits)
    return out[:, 0]


@functools.cache
def _sc_gather_kernel():
    mesh = plsc.VectorSubcoreMesh(core_axis_name="c", subcore_axis_name="s")

    @functools.partial(
        pl.kernel,
        mesh=mesh,
        out_type=jax.ShapeDtypeStruct((BATCH, SEQ), jnp.float32),
        scratch_types=[
            pltpu.VMEM((SEQ,), jnp.int32),
            pltpu.VMEM((SEQ,), jnp.int32),
            pltpu.VMEM((SEQ,), jnp.float32),
            pltpu.VMEM((SEQ,), jnp.float32),
            pltpu.VMEM((SEQ,), jnp.float32),
            pltpu.SemaphoreType.DMA,
        ],
    )
    def _sc_gather(flat_hbm, tokens_hbm, lse_hbm, out_hbm, tok_v, idx_v,
                   val_v, lse_v, o_v, sem):
        b = lax.axis_index("s") * 2 + lax.axis_index("c")
        pltpu.sync_copy(tokens_hbm.at[b], tok_v)
        pltpu.sync_copy(lse_hbm, lse_v)
        for c in range(SEQ // 16):
            tok = tok_v[pl.ds(c * 16, 16)]
            s_vec = lax.iota(jnp.int32, 16) + c * 16
            idx_v[pl.ds(c * 16, 16)] = s_vec * VOCAB + tok
        pltpu.async_copy(flat_hbm.at[idx_v], val_v, sem).wait()
        for c in range(SEQ // 16):
            o_v[pl.ds(c * 16, 16)] = (
                val_v[pl.ds(c * 16, 16)] - lse_v[pl.ds(c * 16, 16)]
            )
        pltpu.sync_copy(o_v, out_hbm.at[b])

    return _sc_gather


def kernel(tokens, logits):
    lse = _lse(logits)
    flat = logits.reshape(SEQ * VOCAB)
    return _sc_gather_kernel()(flat, tokens, lse)


# speedup vs baseline: 1.0029x; 1.0029x over previous
import functools

import jax
import jax.numpy as jnp
from jax import lax
from jax.experimental import pallas as pl
from jax.experimental.pallas import tpu as pltpu
from jax.experimental.pallas import tpu_sc as plsc

SEQ = 32
VOCAB = 1_000_000
BATCH = 32
CHUNK = 65536
NCHUNK = -(-VOCAB // CHUNK)


def _lse_body(x_ref, out_ref, m_ref, s_ref):
    i = pl.program_id(0)
    x = x_ref[...]
    col = lax.broadcasted_iota(jnp.int32, (SEQ, CHUNK), 1) + i * CHUNK
    x = jnp.where(col < VOCAB, x, -jnp.inf)
    mc = jnp.max(x, axis=1, keepdims=True)

    @pl.when(i == 0)
    def _():
        m_ref[...] = jnp.broadcast_to(mc, (SEQ, 128))
        sc = jnp.sum(jnp.exp(x - mc), axis=1, keepdims=True)
        s_ref[...] = jnp.broadcast_to(sc, (SEQ, 128))

    @pl.when(i > 0)
    def _():
        m_old = m_ref[:, :1]
        m_new = jnp.maximum(m_old, mc)
        s_new = s_ref[:, :1] * jnp.exp(m_old - m_new) + jnp.sum(
            jnp.exp(x - m_new), axis=1, keepdims=True
        )
        m_ref[...] = jnp.broadcast_to(m_new, (SEQ, 128))
        s_ref[...] = jnp.broadcast_to(s_new, (SEQ, 128))

    @pl.when(i == NCHUNK - 1)
    def _():
        out_ref[...] = m_ref[...] + jnp.log(s_ref[...])


def _lse(logits):
    return pl.pallas_call(
        _lse_body,
        grid=(NCHUNK,),
        in_specs=[pl.BlockSpec((SEQ, CHUNK), lambda i: (0, i))],
        out_specs=pl.BlockSpec((SEQ, 128), lambda i: (0, 0)),
        out_shape=jax.ShapeDtypeStruct((SEQ, 128), jnp.float32),
        scratch_shapes=[
            pltpu.VMEM((SEQ, 128), jnp.float32),
            pltpu.VMEM((SEQ, 128), jnp.float32),
        ],
    )(logits)


@functools.cache
def _sc_gather_kernel():
    mesh = plsc.VectorSubcoreMesh(core_axis_name="c", subcore_axis_name="s")

    @functools.partial(
        pl.kernel,
        mesh=mesh,
        out_type=jax.ShapeDtypeStruct((BATCH, SEQ), jnp.float32),
        scratch_types=[
            pltpu.VMEM((SEQ,), jnp.int32),
            pltpu.VMEM((SEQ,), jnp.int32),
            pltpu.VMEM((SEQ,), jnp.float32),
            pltpu.VMEM((SEQ,), jnp.float32),
            pltpu.VMEM((SEQ,), jnp.float32),
            pltpu.SemaphoreType.DMA,
        ],
    )
    def _sc_gather(flat_hbm, tokens_hbm, lse_hbm, out_hbm, tok_v, idx_v,
                   val_v, lse_v, o_v, sem):
        b = lax.axis_index("s") * 2 + lax.axis_index("c")
        pltpu.sync_copy(tokens_hbm.at[b], tok_v)
        pltpu.sync_copy(lse_hbm.at[b, pl.ds(0, SEQ)], lse_v)
        for c in range(SEQ // 16):
            tok = tok_v[pl.ds(c * 16, 16)]
            idx_v[pl.ds(c * 16, 16)] = tok + b * VOCAB
        pltpu.async_copy(flat_hbm.at[idx_v], val_v, sem).wait()
        for c in range(SEQ // 16):
            o_v[pl.ds(c * 16, 16)] = (
                val_v[pl.ds(c * 16, 16)] - lse_v[pl.ds(c * 16, 16)]
            )
        pltpu.sync_copy(o_v, out_hbm.at[b])

    return _sc_gather


def kernel(tokens, logits):
    lse = _lse(logits)
    flat = logits.reshape(SEQ * VOCAB)
    return _sc_gather_kernel()(flat, tokens, lse)
